# baseline (speedup 1.0000x reference)
import functools

import jax
import jax.numpy as jnp
from jax import lax
from jax.experimental import pallas as pl
from jax.experimental.pallas import tpu as pltpu
from jax.experimental.pallas import tpu_sc as plsc

_SEQ = 200
_D = 64
_B = 4096
_L = 16
_VOCAB = 1000000
_NTC = 7813
_VPAD = _NTC * 128


@functools.lru_cache(maxsize=None)
def _build_fmt():
    info = plsc.get_sparse_core_info()
    nw = info.num_cores * info.num_subcores
    assert nw == 32
    mesh = plsc.VectorSubcoreMesh(core_axis_name="c", subcore_axis_name="s")
    n_even = _NTC // nw
    n_extra = _NTC - n_even * nw

    @functools.partial(
        pl.kernel,
        mesh=mesh,
        out_type=jax.ShapeDtypeStruct((_VPAD * _D,), jnp.float32),
        scratch_types=[
            pltpu.VMEM((_D, 128), jnp.float32),
            pltpu.VMEM((_D, 128), jnp.float32),
            pltpu.VMEM((128 * _D,), jnp.float32),
            pltpu.VMEM((128 * _D,), jnp.float32),
            pltpu.SemaphoreType.DMA,
            pltpu.SemaphoreType.DMA,
            pltpu.SemaphoreType.DMA,
            pltpu.SemaphoreType.DMA,
        ],
        compiler_params=pltpu.CompilerParams(
            use_tc_tiling_on_sc=True, needs_layout_passes=False),
    )
    def fmt(tt_hbm, out_hbm, in0, in1, ob0, ob1, r0, r1, w0, w1):
        wid = lax.axis_index("s") * info.num_cores + lax.axis_index("c")
        start = wid * n_even + lax.min(wid, n_extra)
        inb = (in0, in1)
        obb = (ob0, ob1)
        rsem = (r0, r1)
        wsem = (w0, w1)

        def read(b, c):
            return pltpu.make_async_copy(
                tt_hbm.at[:, pl.ds(c * 128, 128)], inb[b], rsem[b])

        def write(b, c):
            return pltpu.make_async_copy(
                obb[b], out_hbm.at[pl.ds(c * (128 * _D), 128 * _D)], wsem[b])

        bases = [((g * 16 + lax.iota(jnp.int32, _L)) * _D) for g in range(8)]

        def emit(b):
            @plsc.parallel_loop(0, _D, unroll=4)
            def row(d):
                for g in range(8):
                    plsc.store_scatter(obb[b], [bases[g] + d],
                                       inb[b][d, pl.ds(g * 16, 16)])

        def step(b, k):
            c = start + k
            read(b, c).wait()

            @pl.when(k + 1 < n_even)
            def _():
                read(1 - b, start + k + 1).start()

            @pl.when(k >= 2)
            def _():
                write(b, c).wait()

            emit(b)
            write(b, c).start()

        read(0, start).start()

        def pair(t, carry):
            step(0, 2 * t)
            step(1, 2 * t + 1)
            return carry

        lax.fori_loop(0, n_even // 2, pair, 0)
        write(0, 0).wait()
        write(1, 0).wait()

        @pl.when(wid < n_extra)
        def _():
            c = start + n_even
            read(0, c).start()
            read(0, c).wait()
            emit(0)
            write(0, c).start()
            write(0, c).wait()

    return fmt


@functools.lru_cache(maxsize=None)
def _build_gather():
    info = plsc.get_sparse_core_info()
    nw = info.num_cores * info.num_subcores
    n_blocks = _SEQ * (_B // 128)
    per_w = n_blocks // nw
    bcols = _B // 128
    mesh = plsc.VectorSubcoreMesh(core_axis_name="c", subcore_axis_name="s")

    @functools.partial(
        pl.kernel,
        mesh=mesh,
        out_type=jax.ShapeDtypeStruct((_B * _SEQ * _D,), jnp.float32),
        scratch_types=[
            pltpu.VMEM((2, 128), jnp.int32),
            pltpu.VMEM((128, _D), jnp.float32),
            pltpu.VMEM((128, _D), jnp.float32),
            pltpu.VMEM((_D * 128,), jnp.float32),
            pltpu.VMEM((_D * 128,), jnp.float32),
            pltpu.VMEM((_SEQ * _D,), jnp.float32),
            pltpu.SemaphoreType.DMA,
            pltpu.SemaphoreType.DMA,
            pltpu.SemaphoreType.DMA,
            pltpu.SemaphoreType.DMA,
            pltpu.SemaphoreType.DMA,
            pltpu.SemaphoreType.DMA,
        ],
        compiler_params=pltpu.CompilerParams(
            use_tc_tiling_on_sc=False, needs_layout_passes=False),
    )
    def emb(x_hbm, table_hbm, pos_hbm, out_hbm,
            idx_v, rows0, rows1, ob0, ob1, pos_v,
            g0, g1, i0, i1, o0, o1):
        wid = lax.axis_index("s") * info.num_cores + lax.axis_index("c")
        rows = (rows0, rows1)
        obb = (ob0, ob1)
        gsem = (g0, g1)
        isem = (i0, i1)
        osem = (o0, o1)
        base_blk = wid * per_w

        pltpu.sync_copy(pos_hbm, pos_v)

        def loc(t):
            blk = base_blk + t
            return blk // bcols, blk % bcols

        def idx_copy(b, t):
            l, b0 = loc(t)
            return pltpu.make_async_copy(
                x_hbm.at[pl.ds(l * _B + b0 * 128, 128)], idx_v.at[b], isem[b])

        def gather(b):
            return pltpu.make_async_copy(
                table_hbm.at[idx_v.at[b]], rows[b], gsem[b])

        def out_copy(b, t, d8):
            l, b0 = loc(t)
            off = ((l * 8 + d8) * bcols + b0) * 1024
            return pltpu.make_async_copy(
                obb[b].at[pl.ds(d8 * 1024, 1024)],
                out_hbm.at[pl.ds(off, 1024)], osem[b])

        rvecs = [(g * 16 + lax.iota(jnp.int32, _L)) for g in range(8)]
        zeros16 = jnp.zeros((_L,), jnp.int32)

        def emit(b, t):
            l, _ = loc(t)

            @plsc.parallel_loop(0, _D, unroll=8)
            def col(d):
                cvec = zeros16 + d
                pv = plsc.load_gather(pos_v, [zeros16 + (l * _D + d)])
                for g in range(8):
                    v = plsc.load_gather(rows[b], [rvecs[g], cvec]) + pv
                    obb[b][pl.ds(d * 128 + g * 16, 16)] = v

        def step(b, t):
            gather(b).wait()

            @pl.when(t + 2 < per_w)
            def _():
                idx_copy(b, t + 2).start()

            @pl.when(t + 1 < per_w)
            def _():
                idx_copy(1 - b, t + 1).wait()
                gather(1 - b).start()

            @pl.when(t >= 2)
            def _():
                for d8 in range(8):
                    out_copy(b, t - 2, d8).wait()

            emit(b, t)
            for d8 in range(8):
                out_copy(b, t, d8).start()

        idx_copy(0, 0).start()
        idx_copy(1, 1).start()
        idx_copy(0, 0).wait()
        gather(0).start()

        def pair(u, carry):
            step(0, 2 * u)
            step(1, 2 * u + 1)
            return carry

        lax.fori_loop(0, per_w // 2, pair, 0)
        for d8 in range(8):
            out_copy(0, per_w - 2, d8).wait()
            out_copy(1, per_w - 1, d8).wait()

    return emb


def kernel(x, token_table, pos_table):
    b, l = x.shape
    xflat = x.T.reshape(-1)
    table_rm = _build_fmt()(token_table.T)
    table64 = table_rm.reshape(_VPAD, _D)
    outf = _build_gather()(xflat, table64, pos_table.reshape(-1))
    out5 = outf.reshape(_SEQ, 8, _B // 128, 8, 128)
    return out5.transpose(2, 4, 0, 1, 3).reshape(b, l, _D)


# speedup vs baseline: 1.2518x; 1.2518x over previous
import functools

import jax
import jax.numpy as jnp
from jax import lax
from jax.experimental import pallas as pl
from jax.experimental.pallas import tpu as pltpu
from jax.experimental.pallas import tpu_sc as plsc

_SEQ = 200
_D = 64
_LANES = 16
_CHUNK = 2 * _SEQ


@functools.lru_cache(maxsize=None)
def _build(n_rows: int):
    info = plsc.get_sparse_core_info()
    nw = info.num_cores * info.num_subcores
    assert n_rows % (nw * _CHUNK) == 0
    per_w = n_rows // nw
    n_chunks = per_w // _CHUNK
    assert n_chunks % 2 == 0

    mesh = plsc.VectorSubcoreMesh(core_axis_name="c", subcore_axis_name="s")

    @functools.partial(
        pl.kernel,
        mesh=mesh,
        out_type=jax.ShapeDtypeStruct((n_rows, _D), jnp.float32),
        scratch_types=[
            pltpu.VMEM((2, _CHUNK), jnp.int32),
            pltpu.VMEM((_CHUNK, _D), jnp.float32),
            pltpu.VMEM((_CHUNK, _D), jnp.float32),
            pltpu.VMEM((_CHUNK, _D), jnp.float32),
            pltpu.SemaphoreType.DMA,
            pltpu.SemaphoreType.DMA,
            pltpu.SemaphoreType.DMA,
            pltpu.SemaphoreType.DMA,
        ],
        compiler_params=pltpu.CompilerParams(use_tc_tiling_on_sc=False),
    )
    def emb(x_hbm, table_hbm, pos_hbm, out_hbm,
            idx_v, rows0_v, rows1_v, pos2_v,
            gsem0, gsem1, isem0, isem1):
        wid = lax.axis_index("s") * info.num_cores + lax.axis_index("c")
        base = wid * per_w
        rows = (rows0_v, rows1_v)
        gsem = (gsem0, gsem1)
        isem = (isem0, isem1)

        pltpu.sync_copy(pos_hbm, pos2_v.at[pl.ds(0, _SEQ)])
        pltpu.sync_copy(pos_hbm, pos2_v.at[pl.ds(_SEQ, _SEQ)])

        def idx_copy(b, g):
            return pltpu.make_async_copy(
                x_hbm.at[pl.ds(base + g * _CHUNK, _CHUNK)],
                idx_v.at[b], isem[b])

        def gather(b, g):
            del g
            return pltpu.make_async_copy(
                table_hbm.at[idx_v.at[b]], rows[b], gsem[b])

        idx_copy(0, 0).start()
        idx_copy(1, 1).start()
        idx_copy(0, 0).wait()
        gather(0, 0).start()

        def step(b, g):
            gather(b, g).wait()

            @pl.when(g + 2 < n_chunks)
            def _():
                idx_copy(b, g + 2).start()

            @pl.when(g + 1 < n_chunks)
            def _():
                idx_copy(1 - b, g + 1).wait()
                gather(1 - b, g + 1).start()

            def add_row(i, c):
                for j in range(_D // _LANES):
                    plsc.addupdate(
                        rows[b].at[i, pl.ds(j * _LANES, _LANES)],
                        pos2_v[i, pl.ds(j * _LANES, _LANES)],
                    )
                return c

            lax.fori_loop(0, _CHUNK, add_row, 0, unroll=4)
            pltpu.sync_copy(rows[b], out_hbm.at[pl.ds(base + g * _CHUNK, _CHUNK)])

        def pair(t, carry):
            step(0, 2 * t)
            step(1, 2 * t + 1)
            return carry

        lax.fori_loop(0, n_chunks // 2, pair, 0)

    return emb


def kernel(x, token_table, pos_table):
    b, l = x.shape
    out = _build(b * l)(x.reshape(-1), token_table, pos_table)
    return out.reshape(b, l, _D)
